# baseline (speedup 1.0000x reference)
import functools

import jax
import jax.numpy as jnp
from jax.experimental import pallas as pl
from jax.experimental.pallas import tpu as pltpu


def _leaky(v):
    return jnp.where(v > 0, v, 0.01 * v)


def _pool_kernel(ids_ref, x_ref, wg_ref, bg_ref, wf_ref, bf_ref, wt_ref,
                 bt_ref, xg_ref, out_ref, acc_ref, *, n, nb, ns, d, g, epad):
    t = pl.program_id(0)

    @pl.when(t == 0)
    def _init():
        acc_ref[...] = jnp.zeros_like(acc_ref)

    xb = x_ref[...]
    ids_v = ids_ref[0]
    b = ids_v.shape[1]

    gate = jnp.dot(xb, wg_ref[...],
                   preferred_element_type=jnp.float32) + bg_ref[...]
    e = jnp.exp(gate)
    feat = _leaky(jnp.dot(xb, wf_ref[...],
                          preferred_element_type=jnp.float32) + bf_ref[...])

    parts = [feat[:, i * d:(i + 1) * d] * e[:, i:i + 1] for i in range(ns)]
    e_pad = jnp.concatenate(
        [e, jnp.zeros((b, epad - ns), jnp.float32)], axis=1)
    w_all = jnp.concatenate(parts + [e_pad], axis=1)
    row = t * b + jax.lax.broadcasted_iota(jnp.int32, (b, 1), 0)
    w_all = jnp.where(row < n, w_all, 0.0)

    iota = jax.lax.broadcasted_iota(jnp.int32, (g, b), 0)
    oh = (iota == ids_v).astype(jnp.bfloat16)
    acc_ref[...] += jnp.dot(oh, w_all.astype(jnp.bfloat16),
                            preferred_element_type=jnp.float32)

    @pl.when(t == nb - 1)
    def _epilogue():
        den = acc_ref[:, ns * d:ns * d + ns]
        xg = xg_ref[...]
        for i in range(ns):
            num = acc_ref[:, i * d:(i + 1) * d]
            agg = num / (den[:, i:i + 1] + 1e-16)
            h = (jnp.dot(agg, wt_ref[i, :d, :],
                         preferred_element_type=jnp.float32)
                 + jnp.dot(xg, wt_ref[i, d:, :],
                           preferred_element_type=jnp.float32)
                 + bt_ref[i:i + 1, :])
            xg = _leaky(h) + xg
        out_ref[...] = xg


def kernel(x, x_global, edge_attr, edge_index, batch_ind, num_graphs,
           Wg, bg, Wf, bf, Wt, bt):
    del edge_attr, edge_index, num_graphs
    n, d = x.shape
    g = x_global.shape[0]
    ns = Wg.shape[0]
    bsz = 1024
    nb = pl.cdiv(n, bsz)
    npad = nb * bsz
    epad = 128
    width = ns * d + epad

    ids = jnp.full((npad,), -1, jnp.int32).at[:n].set(batch_ind)
    ids3 = ids.reshape(nb, 1, bsz)
    wg_all = jnp.transpose(Wg[:, :, 0])
    bg_all = bg[:, 0].reshape(1, ns)
    wf_all = jnp.transpose(Wf, (1, 0, 2)).reshape(d, ns * d)
    bf_all = bf.reshape(1, ns * d)

    body = functools.partial(_pool_kernel, n=n, nb=nb, ns=ns, d=d, g=g,
                             epad=epad)
    xg = pl.pallas_call(
        body,
        grid=(nb,),
        in_specs=[
            pl.BlockSpec((1, 1, bsz), lambda t: (t, 0, 0)),
            pl.BlockSpec((bsz, d), lambda t: (t, 0)),
            pl.BlockSpec(wg_all.shape, lambda t: (0, 0)),
            pl.BlockSpec(bg_all.shape, lambda t: (0, 0)),
            pl.BlockSpec(wf_all.shape, lambda t: (0, 0)),
            pl.BlockSpec(bf_all.shape, lambda t: (0, 0)),
            pl.BlockSpec(Wt.shape, lambda t: (0, 0, 0)),
            pl.BlockSpec(bt.shape, lambda t: (0, 0)),
            pl.BlockSpec((g, d), lambda t: (0, 0)),
        ],
        out_specs=pl.BlockSpec((g, d), lambda t: (0, 0)),
        out_shape=jax.ShapeDtypeStruct((g, d), jnp.float32),
        scratch_shapes=[pltpu.VMEM((g, width), jnp.float32)],
        compiler_params=pltpu.CompilerParams(
            dimension_semantics=("arbitrary",)),
    )(ids3, x, wg_all, bg_all, wf_all, bf_all, Wt, bt, x_global)
    return (x, xg)


# speedup vs baseline: 1.3366x; 1.3366x over previous
import functools

import jax
import jax.numpy as jnp
from jax.experimental import pallas as pl
from jax.experimental.pallas import tpu as pltpu


def _leaky(v):
    return jnp.where(v > 0, v, 0.01 * v)


def _pool_kernel(base_ref, ids_ref, x_ref, wg_ref, bg_ref, wf_ref, bf_ref,
                 wt_ref, bt_ref, xg_ref, out_ref, acc_ref, *, n, nb, ns, d,
                 g, epad, win):
    t = pl.program_id(0)

    @pl.when(t == 0)
    def _init():
        acc_ref[...] = jnp.zeros_like(acc_ref)

    xb = x_ref[...]
    ids_v = ids_ref[0]
    b = ids_v.shape[1]

    gate = jnp.dot(xb, wg_ref[...],
                   preferred_element_type=jnp.float32) + bg_ref[...]
    e = jnp.exp(gate)
    feat = _leaky(jnp.dot(xb, wf_ref[...],
                          preferred_element_type=jnp.float32) + bf_ref[...])

    parts = [feat[:, i * d:(i + 1) * d] * e[:, i:i + 1] for i in range(ns)]
    e_pad = jnp.concatenate(
        [e, jnp.zeros((b, epad - ns), jnp.float32)], axis=1)
    w_all = jnp.concatenate(parts + [e_pad], axis=1)
    row = t * b + jax.lax.broadcasted_iota(jnp.int32, (b, 1), 0)
    w_all = jnp.where(row < n, w_all, 0.0)

    w16 = w_all.astype(jnp.bfloat16)
    base = pl.multiple_of(base_ref[t], 8)
    rel = ids_v - base
    iota_w = jax.lax.broadcasted_iota(jnp.int32, (win, b), 0)
    oh = (iota_w == rel).astype(jnp.bfloat16)
    acc_ref[pl.ds(base, win), :] += jnp.dot(
        oh, w16, preferred_element_type=jnp.float32)

    overflow = rel >= win

    @pl.when(jnp.any(overflow))
    def _spill():
        iota_g = jax.lax.broadcasted_iota(jnp.int32, (g, b), 0)
        oh_full = ((iota_g == ids_v) & overflow).astype(jnp.bfloat16)
        acc_ref[:g, :] += jnp.dot(oh_full, w16,
                                  preferred_element_type=jnp.float32)

    @pl.when(t == nb - 1)
    def _epilogue():
        den = acc_ref[:g, ns * d:ns * d + ns]
        xg = xg_ref[...]
        for i in range(ns):
            num = acc_ref[:g, i * d:(i + 1) * d]
            agg = num / (den[:, i:i + 1] + 1e-16)
            h = (jnp.dot(agg, wt_ref[i, :d, :],
                         preferred_element_type=jnp.float32)
                 + jnp.dot(xg, wt_ref[i, d:, :],
                           preferred_element_type=jnp.float32)
                 + bt_ref[i:i + 1, :])
            xg = _leaky(h) + xg
        out_ref[...] = xg


def kernel(x, x_global, edge_attr, edge_index, batch_ind, num_graphs,
           Wg, bg, Wf, bf, Wt, bt):
    del edge_attr, edge_index, num_graphs
    n, d = x.shape
    g = x_global.shape[0]
    ns = Wg.shape[0]
    bsz = 1024
    nb = pl.cdiv(n, bsz)
    npad = nb * bsz
    epad = 128
    width = ns * d + epad
    win = 136

    ids = jnp.full((npad,), -1, jnp.int32).at[:n].set(batch_ind)
    ids3 = ids.reshape(nb, 1, bsz)
    bases = (batch_ind[::bsz] // 8) * 8
    wg_all = jnp.transpose(Wg[:, :, 0])
    bg_all = bg[:, 0].reshape(1, ns)
    wf_all = jnp.transpose(Wf, (1, 0, 2)).reshape(d, ns * d)
    bf_all = bf.reshape(1, ns * d)

    body = functools.partial(_pool_kernel, n=n, nb=nb, ns=ns, d=d, g=g,
                             epad=epad, win=win)
    xg = pl.pallas_call(
        body,
        grid=(nb,),
        in_specs=[
            pl.BlockSpec(memory_space=pltpu.SMEM),
            pl.BlockSpec((1, 1, bsz), lambda t: (t, 0, 0)),
            pl.BlockSpec((bsz, d), lambda t: (t, 0)),
            pl.BlockSpec(wg_all.shape, lambda t: (0, 0)),
            pl.BlockSpec(bg_all.shape, lambda t: (0, 0)),
            pl.BlockSpec(wf_all.shape, lambda t: (0, 0)),
            pl.BlockSpec(bf_all.shape, lambda t: (0, 0)),
            pl.BlockSpec(Wt.shape, lambda t: (0, 0, 0)),
            pl.BlockSpec(bt.shape, lambda t: (0, 0)),
            pl.BlockSpec((g, d), lambda t: (0, 0)),
        ],
        out_specs=pl.BlockSpec((g, d), lambda t: (0, 0)),
        out_shape=jax.ShapeDtypeStruct((g, d), jnp.float32),
        scratch_shapes=[pltpu.VMEM((g + win, width), jnp.float32)],
        compiler_params=pltpu.CompilerParams(
            dimension_semantics=("arbitrary",)),
    )(bases, ids3, x, wg_all, bg_all, wf_all, bf_all, Wt, bt, x_global)
    return (x, xg)


# speedup vs baseline: 1.3727x; 1.0270x over previous
import functools

import jax
import jax.numpy as jnp
from jax.experimental import pallas as pl
from jax.experimental.pallas import tpu as pltpu


def _leaky(v):
    return jnp.maximum(v, 0.01 * v)


def _pool_kernel(base_ref, ids_ref, x_ref, wg_ref, bg_ref, wf_ref, bf_ref,
                 rep_ref, wt_ref, bt_ref, xg_ref, out_ref, acc_ref, *, n,
                 nb, ns, d, g, epad, win):
    t = pl.program_id(0)

    @pl.when(t == 0)
    def _init():
        acc_ref[...] = jnp.zeros_like(acc_ref)

    ids_v = ids_ref[0]
    b = ids_v.shape[1]
    row = t * b + jax.lax.broadcasted_iota(jnp.int32, (b, 1), 0)
    xb = jnp.where(row < n, x_ref[...], 0.0)

    gate = jnp.dot(xb, wg_ref[...],
                   preferred_element_type=jnp.float32) + bg_ref[...]
    e16 = jnp.exp(gate).astype(jnp.bfloat16)
    feat16 = _leaky(
        jnp.dot(xb, wf_ref[...], preferred_element_type=jnp.float32)
        + bf_ref[...]).astype(jnp.bfloat16)
    e_rep = jnp.dot(e16, rep_ref[...],
                    preferred_element_type=jnp.float32).astype(jnp.bfloat16)
    w_feat = feat16 * e_rep
    e_pad = jnp.concatenate(
        [e16, jnp.zeros((b, epad - ns), jnp.bfloat16)], axis=1)
    w16 = jnp.concatenate([w_feat, e_pad], axis=1)
    base = pl.multiple_of(base_ref[t], 8)
    rel = ids_v - base
    iota_w = jax.lax.broadcasted_iota(jnp.int32, (win, b), 0)
    oh = (iota_w == rel).astype(jnp.bfloat16)
    acc_ref[pl.ds(base, win), :] += jnp.dot(
        oh, w16, preferred_element_type=jnp.float32)

    overflow = rel >= win

    @pl.when(jnp.any(overflow))
    def _spill():
        iota_g = jax.lax.broadcasted_iota(jnp.int32, (g, b), 0)
        oh_full = ((iota_g == ids_v) & overflow).astype(jnp.bfloat16)
        acc_ref[:g, :] += jnp.dot(oh_full, w16,
                                  preferred_element_type=jnp.float32)

    @pl.when(t == nb - 1)
    def _epilogue():
        den = acc_ref[:g, ns * d:ns * d + ns]
        xg = xg_ref[...]
        for i in range(ns):
            num = acc_ref[:g, i * d:(i + 1) * d]
            agg = num / (den[:, i:i + 1] + 1e-16)
            h = (jnp.dot(agg, wt_ref[i, :d, :],
                         preferred_element_type=jnp.float32)
                 + jnp.dot(xg, wt_ref[i, d:, :],
                           preferred_element_type=jnp.float32)
                 + bt_ref[i:i + 1, :])
            xg = _leaky(h) + xg
        out_ref[...] = xg


def kernel(x, x_global, edge_attr, edge_index, batch_ind, num_graphs,
           Wg, bg, Wf, bf, Wt, bt):
    del edge_attr, edge_index, num_graphs
    n, d = x.shape
    g = x_global.shape[0]
    ns = Wg.shape[0]
    bsz = 1024
    nb = pl.cdiv(n, bsz)
    npad = nb * bsz
    epad = 128
    width = ns * d + epad
    win = 136

    ids = jnp.full((npad,), -1, jnp.int32).at[:n].set(batch_ind)
    ids3 = ids.reshape(nb, 1, bsz)
    bases = (batch_ind[::bsz] // 8) * 8
    rep = (jnp.arange(ns * d, dtype=jnp.int32)[None, :] // d
           == jnp.arange(ns, dtype=jnp.int32)[:, None]
           ).astype(jnp.bfloat16)
    wg_all = jnp.transpose(Wg[:, :, 0])
    bg_all = bg[:, 0].reshape(1, ns)
    wf_all = jnp.transpose(Wf, (1, 0, 2)).reshape(d, ns * d)
    bf_all = bf.reshape(1, ns * d)

    body = functools.partial(_pool_kernel, n=n, nb=nb, ns=ns, d=d, g=g,
                             epad=epad, win=win)
    xg = pl.pallas_call(
        body,
        grid=(nb,),
        in_specs=[
            pl.BlockSpec(memory_space=pltpu.SMEM),
            pl.BlockSpec((1, 1, bsz), lambda t: (t, 0, 0)),
            pl.BlockSpec((bsz, d), lambda t: (t, 0)),
            pl.BlockSpec(wg_all.shape, lambda t: (0, 0)),
            pl.BlockSpec(bg_all.shape, lambda t: (0, 0)),
            pl.BlockSpec(wf_all.shape, lambda t: (0, 0)),
            pl.BlockSpec(bf_all.shape, lambda t: (0, 0)),
            pl.BlockSpec(rep.shape, lambda t: (0, 0)),
            pl.BlockSpec(Wt.shape, lambda t: (0, 0, 0)),
            pl.BlockSpec(bt.shape, lambda t: (0, 0)),
            pl.BlockSpec((g, d), lambda t: (0, 0)),
        ],
        out_specs=pl.BlockSpec((g, d), lambda t: (0, 0)),
        out_shape=jax.ShapeDtypeStruct((g, d), jnp.float32),
        scratch_shapes=[pltpu.VMEM((g + win, width), jnp.float32)],
        compiler_params=pltpu.CompilerParams(
            dimension_semantics=("arbitrary",)),
    )(bases, ids3, x, wg_all, bg_all, wf_all, bf_all, rep, Wt, bt,
      x_global)
    return (x, xg)


# speedup vs baseline: 1.4470x; 1.0541x over previous
import functools

import jax
import jax.numpy as jnp
from jax.experimental import pallas as pl
from jax.experimental.pallas import tpu as pltpu


def _leaky(v):
    return jnp.maximum(v, 0.01 * v)


def _pool_kernel(base_ref, ids_ref, x_ref, wg_ref, bg_ref, wf_ref, bf_ref,
                 rep_ref, wt_ref, bt_ref, xg_ref, out_ref, acc_ref, *, n,
                 nb, ns, d, g, epad, win):
    t = pl.program_id(0)

    @pl.when(t == 0)
    def _init():
        acc_ref[...] = jnp.zeros_like(acc_ref)

    ids_v = ids_ref[0]
    b = ids_v.shape[1]
    row = t * b + jax.lax.broadcasted_iota(jnp.int32, (b, 1), 0)
    xb = jnp.where(row < n, x_ref[...], 0.0).astype(jnp.bfloat16)

    gate = jnp.dot(xb, wg_ref[...],
                   preferred_element_type=jnp.float32) + bg_ref[...]
    e16 = jnp.exp(gate).astype(jnp.bfloat16)
    feat16 = _leaky(
        jnp.dot(xb, wf_ref[...],
                preferred_element_type=jnp.float32).astype(jnp.bfloat16)
        + bf_ref[...])
    e_rep = jnp.dot(e16, rep_ref[...],
                    preferred_element_type=jnp.float32).astype(jnp.bfloat16)
    w_feat = feat16 * e_rep
    e_pad = jnp.concatenate(
        [e16, jnp.zeros((b, epad - ns), jnp.bfloat16)], axis=1)
    w16 = jnp.concatenate([w_feat, e_pad], axis=1)
    base = pl.multiple_of(base_ref[t], 8)
    rel = ids_v - base
    iota_w = jax.lax.broadcasted_iota(jnp.int32, (win, b), 0)
    oh = (iota_w == rel).astype(jnp.bfloat16)
    acc_ref[pl.ds(base, win), :] += jnp.dot(
        oh, w16, preferred_element_type=jnp.float32)

    overflow = rel >= win

    @pl.when(jnp.any(overflow))
    def _spill():
        iota_g = jax.lax.broadcasted_iota(jnp.int32, (g, b), 0)
        oh_full = ((iota_g == ids_v) & overflow).astype(jnp.bfloat16)
        acc_ref[:g, :] += jnp.dot(oh_full, w16,
                                  preferred_element_type=jnp.float32)

    @pl.when(t == nb - 1)
    def _epilogue():
        den = acc_ref[:g, ns * d:ns * d + ns]
        xg = xg_ref[...]
        for i in range(ns):
            num = acc_ref[:g, i * d:(i + 1) * d]
            agg = num / (den[:, i:i + 1] + 1e-16)
            h = (jnp.dot(agg, wt_ref[i, :d, :],
                         preferred_element_type=jnp.float32)
                 + jnp.dot(xg, wt_ref[i, d:, :],
                           preferred_element_type=jnp.float32)
                 + bt_ref[i:i + 1, :])
            xg = _leaky(h) + xg
        out_ref[...] = xg


def kernel(x, x_global, edge_attr, edge_index, batch_ind, num_graphs,
           Wg, bg, Wf, bf, Wt, bt):
    del edge_attr, edge_index, num_graphs
    n, d = x.shape
    g = x_global.shape[0]
    ns = Wg.shape[0]
    bsz = 1024
    nb = pl.cdiv(n, bsz)
    npad = nb * bsz
    epad = 128
    width = ns * d + epad
    win = 40

    ids = jnp.full((npad,), -1, jnp.int32).at[:n].set(batch_ind)
    ids3 = ids.reshape(nb, 1, bsz)
    bases = (batch_ind[::bsz] // 8) * 8
    rep = (jnp.arange(ns * d, dtype=jnp.int32)[None, :] // d
           == jnp.arange(ns, dtype=jnp.int32)[:, None]
           ).astype(jnp.bfloat16)
    wg_all = jnp.transpose(Wg[:, :, 0]).astype(jnp.bfloat16)
    bg_all = bg[:, 0].reshape(1, ns)
    wf_all = jnp.transpose(Wf, (1, 0, 2)).reshape(d, ns * d).astype(
        jnp.bfloat16)
    bf_all = bf.reshape(1, ns * d).astype(jnp.bfloat16)

    body = functools.partial(_pool_kernel, n=n, nb=nb, ns=ns, d=d, g=g,
                             epad=epad, win=win)
    xg = pl.pallas_call(
        body,
        grid=(nb,),
        in_specs=[
            pl.BlockSpec(memory_space=pltpu.SMEM),
            pl.BlockSpec((1, 1, bsz), lambda t: (t, 0, 0)),
            pl.BlockSpec((bsz, d), lambda t: (t, 0)),
            pl.BlockSpec(wg_all.shape, lambda t: (0, 0)),
            pl.BlockSpec(bg_all.shape, lambda t: (0, 0)),
            pl.BlockSpec(wf_all.shape, lambda t: (0, 0)),
            pl.BlockSpec(bf_all.shape, lambda t: (0, 0)),
            pl.BlockSpec(rep.shape, lambda t: (0, 0)),
            pl.BlockSpec(Wt.shape, lambda t: (0, 0, 0)),
            pl.BlockSpec(bt.shape, lambda t: (0, 0)),
            pl.BlockSpec((g, d), lambda t: (0, 0)),
        ],
        out_specs=pl.BlockSpec((g, d), lambda t: (0, 0)),
        out_shape=jax.ShapeDtypeStruct((g, d), jnp.float32),
        scratch_shapes=[pltpu.VMEM((g + win, width), jnp.float32)],
        compiler_params=pltpu.CompilerParams(
            dimension_semantics=("arbitrary",)),
    )(bases, ids3, x, wg_all, bg_all, wf_all, bf_all, rep, Wt, bt,
      x_global)
    return (x, xg)


# speedup vs baseline: 1.4474x; 1.0003x over previous
import functools

import jax
import jax.numpy as jnp
from jax.experimental import pallas as pl
from jax.experimental.pallas import tpu as pltpu


def _leaky(v):
    return jnp.maximum(v, 0.01 * v)


def _pool_kernel(base_ref, ids_ref, x_ref, wg_ref, bg_ref, wf_ref, bf_ref,
                 rep_ref, wt_ref, bt_ref, xg_ref, out_ref, acc_ref,
                 den_ref, *, n, nb, ns, d, g, epad, win):
    t = pl.program_id(0)

    @pl.when(t == 0)
    def _init():
        acc_ref[...] = jnp.zeros_like(acc_ref)
        den_ref[...] = jnp.zeros_like(den_ref)

    ids_v = ids_ref[0]
    b = ids_v.shape[1]
    row = t * b + jax.lax.broadcasted_iota(jnp.int32, (b, 1), 0)
    xb = jnp.where(row < n, x_ref[...], 0.0).astype(jnp.bfloat16)

    gate = jnp.dot(xb, wg_ref[...],
                   preferred_element_type=jnp.float32) + bg_ref[...]
    e16 = jnp.exp(gate).astype(jnp.bfloat16)
    feat16 = _leaky(
        jnp.dot(xb, wf_ref[...],
                preferred_element_type=jnp.float32).astype(jnp.bfloat16)
        + bf_ref[...])
    e_rep = jnp.dot(e16, rep_ref[...],
                    preferred_element_type=jnp.float32).astype(jnp.bfloat16)
    w_feat = feat16 * e_rep
    e_pad = jnp.concatenate(
        [e16, jnp.zeros((b, epad - ns), jnp.bfloat16)], axis=1)
    base = pl.multiple_of(base_ref[t], 8)
    rel = ids_v - base
    iota_w = jax.lax.broadcasted_iota(jnp.int32, (win, b), 0)
    oh = (iota_w == rel).astype(jnp.bfloat16)
    acc_ref[pl.ds(base, win), :] += jnp.dot(
        oh, w_feat, preferred_element_type=jnp.float32)
    den_ref[pl.ds(base, win), :] += jnp.dot(
        oh, e_pad, preferred_element_type=jnp.float32)

    overflow = rel >= win

    @pl.when(jnp.any(overflow))
    def _spill():
        iota_g = jax.lax.broadcasted_iota(jnp.int32, (g, b), 0)
        oh_full = ((iota_g == ids_v) & overflow).astype(jnp.bfloat16)
        acc_ref[:g, :] += jnp.dot(oh_full, w_feat,
                                  preferred_element_type=jnp.float32)
        den_ref[:g, :] += jnp.dot(oh_full, e_pad,
                                  preferred_element_type=jnp.float32)

    @pl.when(t == nb - 1)
    def _epilogue():
        den = den_ref[:g, 0:ns]
        xg = xg_ref[...]
        for i in range(ns):
            num = acc_ref[:g, i * d:(i + 1) * d]
            agg = num / (den[:, i:i + 1] + 1e-16)
            h = (jnp.dot(agg, wt_ref[i, :d, :],
                         preferred_element_type=jnp.float32)
                 + jnp.dot(xg, wt_ref[i, d:, :],
                           preferred_element_type=jnp.float32)
                 + bt_ref[i:i + 1, :])
            xg = _leaky(h) + xg
        out_ref[...] = xg


def kernel(x, x_global, edge_attr, edge_index, batch_ind, num_graphs,
           Wg, bg, Wf, bf, Wt, bt):
    del edge_attr, edge_index, num_graphs
    n, d = x.shape
    g = x_global.shape[0]
    ns = Wg.shape[0]
    bsz = 1024
    nb = pl.cdiv(n, bsz)
    npad = nb * bsz
    epad = 128
    width = ns * d + epad
    win = 40

    ids = jnp.full((npad,), -1, jnp.int32).at[:n].set(batch_ind)
    ids3 = ids.reshape(nb, 1, bsz)
    bases = (batch_ind[::bsz] // 8) * 8
    rep = (jnp.arange(ns * d, dtype=jnp.int32)[None, :] // d
           == jnp.arange(ns, dtype=jnp.int32)[:, None]
           ).astype(jnp.bfloat16)
    wg_all = jnp.transpose(Wg[:, :, 0]).astype(jnp.bfloat16)
    bg_all = bg[:, 0].reshape(1, ns)
    wf_all = jnp.transpose(Wf, (1, 0, 2)).reshape(d, ns * d).astype(
        jnp.bfloat16)
    bf_all = bf.reshape(1, ns * d).astype(jnp.bfloat16)

    body = functools.partial(_pool_kernel, n=n, nb=nb, ns=ns, d=d, g=g,
                             epad=epad, win=win)
    xg = pl.pallas_call(
        body,
        grid=(nb,),
        in_specs=[
            pl.BlockSpec(memory_space=pltpu.SMEM),
            pl.BlockSpec((1, 1, bsz), lambda t: (t, 0, 0)),
            pl.BlockSpec((bsz, d), lambda t: (t, 0)),
            pl.BlockSpec(wg_all.shape, lambda t: (0, 0)),
            pl.BlockSpec(bg_all.shape, lambda t: (0, 0)),
            pl.BlockSpec(wf_all.shape, lambda t: (0, 0)),
            pl.BlockSpec(bf_all.shape, lambda t: (0, 0)),
            pl.BlockSpec(rep.shape, lambda t: (0, 0)),
            pl.BlockSpec(Wt.shape, lambda t: (0, 0, 0)),
            pl.BlockSpec(bt.shape, lambda t: (0, 0)),
            pl.BlockSpec((g, d), lambda t: (0, 0)),
        ],
        out_specs=pl.BlockSpec((g, d), lambda t: (0, 0)),
        out_shape=jax.ShapeDtypeStruct((g, d), jnp.float32),
        scratch_shapes=[pltpu.VMEM((g + win, ns * d), jnp.float32),
                        pltpu.VMEM((g + win, epad), jnp.float32)],
        compiler_params=pltpu.CompilerParams(
            dimension_semantics=("arbitrary",)),
    )(bases, ids3, x, wg_all, bg_all, wf_all, bf_all, rep, Wt, bt,
      x_global)
    return (x, xg)


# speedup vs baseline: 1.5564x; 1.0753x over previous
import functools

import jax
import jax.numpy as jnp
from jax.experimental import pallas as pl
from jax.experimental.pallas import tpu as pltpu


def _leaky(v):
    return jnp.maximum(v, 0.01 * v)


def _pool_kernel(base_ref, ids_ref, x_ref, wg_ref, bg_ref, wf_ref, bf_ref,
                 rep_ref, wt_ref, bt_ref, xg_ref, out_ref, acc_ref,
                 den_ref, *, n, nb, ns, d, g, epad, win):
    t = pl.program_id(0)

    @pl.when(t == 0)
    def _init():
        acc_ref[...] = jnp.zeros_like(acc_ref)
        den_ref[...] = jnp.zeros_like(den_ref)

    ids_v = ids_ref[0]
    b = ids_v.shape[1]
    row = t * b + jax.lax.broadcasted_iota(jnp.int32, (b, 1), 0)
    xb = jnp.where(row < n, x_ref[...], 0.0).astype(jnp.bfloat16)

    gate = jnp.dot(xb, wg_ref[...],
                   preferred_element_type=jnp.float32) + bg_ref[...]
    e16 = jnp.exp(gate).astype(jnp.bfloat16)
    feat16 = _leaky(
        jnp.dot(xb, wf_ref[...],
                preferred_element_type=jnp.float32).astype(jnp.bfloat16)
        + bf_ref[...])
    e_rep = jnp.dot(e16, rep_ref[...],
                    preferred_element_type=jnp.float32).astype(jnp.bfloat16)
    w_feat = feat16 * e_rep
    e_pad = jnp.concatenate(
        [e16, jnp.zeros((b, epad - ns), jnp.bfloat16)], axis=1)
    base = pl.multiple_of(base_ref[t], 8)
    rel = ids_v - base
    iota_w = jax.lax.broadcasted_iota(jnp.int32, (win, b), 0)
    oh = (iota_w == rel).astype(jnp.bfloat16)
    acc_ref[pl.ds(base, win), :] += jnp.dot(
        oh, w_feat, preferred_element_type=jnp.float32)
    den_ref[pl.ds(base, win), :] += jnp.dot(
        oh, e_pad, preferred_element_type=jnp.float32)

    overflow = rel >= win

    @pl.when(jnp.any(overflow))
    def _spill():
        iota_g = jax.lax.broadcasted_iota(jnp.int32, (g, b), 0)
        oh_full = ((iota_g == ids_v) & overflow).astype(jnp.bfloat16)
        acc_ref[:g, :] += jnp.dot(oh_full, w_feat,
                                  preferred_element_type=jnp.float32)
        den_ref[:g, :] += jnp.dot(oh_full, e_pad,
                                  preferred_element_type=jnp.float32)

    @pl.when(t == nb - 1)
    def _epilogue():
        den = den_ref[:g, 0:ns]
        xg = xg_ref[...]
        for i in range(ns):
            num = acc_ref[:g, i * d:(i + 1) * d]
            agg = num / (den[:, i:i + 1] + 1e-16)
            h = (jnp.dot(agg, wt_ref[i, :d, :],
                         preferred_element_type=jnp.float32)
                 + jnp.dot(xg, wt_ref[i, d:, :],
                           preferred_element_type=jnp.float32)
                 + bt_ref[i:i + 1, :])
            xg = _leaky(h) + xg
        out_ref[...] = xg


def kernel(x, x_global, edge_attr, edge_index, batch_ind, num_graphs,
           Wg, bg, Wf, bf, Wt, bt):
    del edge_attr, edge_index, num_graphs
    n, d = x.shape
    g = x_global.shape[0]
    ns = Wg.shape[0]
    bsz = 2048
    nb = pl.cdiv(n, bsz)
    npad = nb * bsz
    epad = 128
    width = ns * d + epad
    win = 40

    ids = jnp.full((npad,), -1, jnp.int32).at[:n].set(batch_ind)
    ids3 = ids.reshape(nb, 1, bsz)
    bases = (batch_ind[::bsz] // 8) * 8
    rep = (jnp.arange(ns * d, dtype=jnp.int32)[None, :] // d
           == jnp.arange(ns, dtype=jnp.int32)[:, None]
           ).astype(jnp.bfloat16)
    wg_all = jnp.transpose(Wg[:, :, 0]).astype(jnp.bfloat16)
    bg_all = bg[:, 0].reshape(1, ns)
    wf_all = jnp.transpose(Wf, (1, 0, 2)).reshape(d, ns * d).astype(
        jnp.bfloat16)
    bf_all = bf.reshape(1, ns * d).astype(jnp.bfloat16)

    body = functools.partial(_pool_kernel, n=n, nb=nb, ns=ns, d=d, g=g,
                             epad=epad, win=win)
    xg = pl.pallas_call(
        body,
        grid=(nb,),
        in_specs=[
            pl.BlockSpec(memory_space=pltpu.SMEM),
            pl.BlockSpec((1, 1, bsz), lambda t: (t, 0, 0)),
            pl.BlockSpec((bsz, d), lambda t: (t, 0)),
            pl.BlockSpec(wg_all.shape, lambda t: (0, 0)),
            pl.BlockSpec(bg_all.shape, lambda t: (0, 0)),
            pl.BlockSpec(wf_all.shape, lambda t: (0, 0)),
            pl.BlockSpec(bf_all.shape, lambda t: (0, 0)),
            pl.BlockSpec(rep.shape, lambda t: (0, 0)),
            pl.BlockSpec(Wt.shape, lambda t: (0, 0, 0)),
            pl.BlockSpec(bt.shape, lambda t: (0, 0)),
            pl.BlockSpec((g, d), lambda t: (0, 0)),
        ],
        out_specs=pl.BlockSpec((g, d), lambda t: (0, 0)),
        out_shape=jax.ShapeDtypeStruct((g, d), jnp.float32),
        scratch_shapes=[pltpu.VMEM((g + win, ns * d), jnp.float32),
                        pltpu.VMEM((g + win, epad), jnp.float32)],
        compiler_params=pltpu.CompilerParams(
            dimension_semantics=("arbitrary",)),
    )(bases, ids3, x, wg_all, bg_all, wf_all, bf_all, rep, Wt, bt,
      x_global)
    return (x, xg)


# speedup vs baseline: 1.6041x; 1.0307x over previous
import functools

import jax
import jax.numpy as jnp
from jax.experimental import pallas as pl
from jax.experimental.pallas import tpu as pltpu


def _leaky(v):
    return jnp.maximum(v, 0.01 * v)


def _pool_kernel(base_ref, ids_ref, x_ref, wg_ref, bg_ref, wf_ref, bf_ref,
                 rep_ref, wt_ref, bt_ref, xg_ref, out_ref, acc_ref,
                 den_ref, *, n, nb, ns, d, g, epad, win):
    t = pl.program_id(0)

    @pl.when(t == 0)
    def _init():
        acc_ref[...] = jnp.zeros_like(acc_ref)
        den_ref[...] = jnp.zeros_like(den_ref)

    ids_v = ids_ref[0]
    b = ids_v.shape[1]
    row = t * b + jax.lax.broadcasted_iota(jnp.int32, (b, 1), 0)
    xb = jnp.where(row < n, x_ref[...], 0.0).astype(jnp.bfloat16)

    gate = jnp.dot(xb, wg_ref[...],
                   preferred_element_type=jnp.float32) + bg_ref[...]
    e16 = jnp.exp(gate).astype(jnp.bfloat16)
    feat16 = _leaky(
        jnp.dot(xb, wf_ref[...],
                preferred_element_type=jnp.float32).astype(jnp.bfloat16)
        + bf_ref[...])
    e_rep = jnp.dot(e16, rep_ref[...],
                    preferred_element_type=jnp.float32).astype(jnp.bfloat16)
    w_feat = feat16 * e_rep
    e_pad = jnp.concatenate(
        [e16, jnp.zeros((b, epad - ns), jnp.bfloat16)], axis=1)
    base = pl.multiple_of(base_ref[t], 8)
    rel = ids_v - base
    iota_w = jax.lax.broadcasted_iota(jnp.int32, (win, b), 0)
    oh = (iota_w == rel).astype(jnp.bfloat16)
    acc_ref[pl.ds(base, win), :] += jnp.dot(
        oh, w_feat, preferred_element_type=jnp.float32)
    den_ref[pl.ds(base, win), :] += jnp.dot(
        oh, e_pad, preferred_element_type=jnp.float32)

    overflow = rel >= win

    @pl.when(jnp.any(overflow))
    def _spill():
        iota_g = jax.lax.broadcasted_iota(jnp.int32, (g, b), 0)
        oh_full = ((iota_g == ids_v) & overflow).astype(jnp.bfloat16)
        acc_ref[:g, :] += jnp.dot(oh_full, w_feat,
                                  preferred_element_type=jnp.float32)
        den_ref[:g, :] += jnp.dot(oh_full, e_pad,
                                  preferred_element_type=jnp.float32)

    @pl.when(t == nb - 1)
    def _epilogue():
        den = den_ref[:g, 0:ns]
        xg = xg_ref[...]
        for i in range(ns):
            num = acc_ref[:g, i * d:(i + 1) * d]
            agg = num / (den[:, i:i + 1] + 1e-16)
            h = (jnp.dot(agg, wt_ref[i, :d, :],
                         preferred_element_type=jnp.float32)
                 + jnp.dot(xg, wt_ref[i, d:, :],
                           preferred_element_type=jnp.float32)
                 + bt_ref[i:i + 1, :])
            xg = _leaky(h) + xg
        out_ref[...] = xg


def kernel(x, x_global, edge_attr, edge_index, batch_ind, num_graphs,
           Wg, bg, Wf, bf, Wt, bt):
    del edge_attr, edge_index, num_graphs
    n, d = x.shape
    g = x_global.shape[0]
    ns = Wg.shape[0]
    bsz = 4096
    nb = pl.cdiv(n, bsz)
    npad = nb * bsz
    epad = 128
    width = ns * d + epad
    win = 48

    ids = jnp.full((npad,), -1, jnp.int32).at[:n].set(batch_ind)
    ids3 = ids.reshape(nb, 1, bsz)
    bases = (batch_ind[::bsz] // 8) * 8
    rep = (jnp.arange(ns * d, dtype=jnp.int32)[None, :] // d
           == jnp.arange(ns, dtype=jnp.int32)[:, None]
           ).astype(jnp.bfloat16)
    wg_all = jnp.transpose(Wg[:, :, 0]).astype(jnp.bfloat16)
    bg_all = bg[:, 0].reshape(1, ns)
    wf_all = jnp.transpose(Wf, (1, 0, 2)).reshape(d, ns * d).astype(
        jnp.bfloat16)
    bf_all = bf.reshape(1, ns * d).astype(jnp.bfloat16)

    body = functools.partial(_pool_kernel, n=n, nb=nb, ns=ns, d=d, g=g,
                             epad=epad, win=win)
    xg = pl.pallas_call(
        body,
        grid=(nb,),
        in_specs=[
            pl.BlockSpec(memory_space=pltpu.SMEM),
            pl.BlockSpec((1, 1, bsz), lambda t: (t, 0, 0)),
            pl.BlockSpec((bsz, d), lambda t: (t, 0)),
            pl.BlockSpec(wg_all.shape, lambda t: (0, 0)),
            pl.BlockSpec(bg_all.shape, lambda t: (0, 0)),
            pl.BlockSpec(wf_all.shape, lambda t: (0, 0)),
            pl.BlockSpec(bf_all.shape, lambda t: (0, 0)),
            pl.BlockSpec(rep.shape, lambda t: (0, 0)),
            pl.BlockSpec(Wt.shape, lambda t: (0, 0, 0)),
            pl.BlockSpec(bt.shape, lambda t: (0, 0)),
            pl.BlockSpec((g, d), lambda t: (0, 0)),
        ],
        out_specs=pl.BlockSpec((g, d), lambda t: (0, 0)),
        out_shape=jax.ShapeDtypeStruct((g, d), jnp.float32),
        scratch_shapes=[pltpu.VMEM((g + win, ns * d), jnp.float32),
                        pltpu.VMEM((g + win, epad), jnp.float32)],
        compiler_params=pltpu.CompilerParams(
            dimension_semantics=("arbitrary",)),
    )(bases, ids3, x, wg_all, bg_all, wf_all, bf_all, rep, Wt, bt,
      x_global)
    return (x, xg)
